# Initial kernel scaffold; baseline (speedup 1.0000x reference)
import jax
import jax.numpy as jnp
from jax.experimental import pallas as pl


def kernel(node_raw, edge_index, geo, cond, W_nproj, b_nproj, W_geo, b_geo, W_cond, b_cond, W_film, b_film):
    raise NotImplementedError("write your pallas kernel here")


# speedup vs baseline: 2.8152x; 2.8152x over previous
import functools

import jax
import jax.numpy as jnp
from jax import lax
from jax.experimental import pallas as pl
from jax.experimental.pallas import tpu as pltpu
from jax.experimental.pallas import tpu_sc as plsc


def _node_proj_body(x_ref, wn_ref, bn_ref, wf1_ref, o_ref):
    nf = jnp.dot(x_ref[...], wn_ref[...], preferred_element_type=jnp.float32)
    nf = jnp.maximum(nf + bn_ref[...], 0.0)
    o_ref[...] = jnp.dot(nf, wf1_ref[...], preferred_element_type=jnp.float32)


def _edge_fuse_body(g_ref, geo_ref, cond_ref, wg_ref, wf2_ref, wc_ref,
                    bg_ref, bf_ref, bc_ref, o_ref):
    eg = jnp.dot(geo_ref[...], wg_ref[...], preferred_element_type=jnp.float32)
    eg = jnp.maximum(eg + bg_ref[...], 0.0)
    h = g_ref[...] + jnp.dot(eg, wf2_ref[...], preferred_element_type=jnp.float32)
    h = h + bf_ref[...]
    mu = jnp.mean(h, axis=-1, keepdims=True)
    hc = h - mu
    var = jnp.mean(hc * hc, axis=-1, keepdims=True)
    hln = hc * lax.rsqrt(var + 1e-5)
    gb = jnp.dot(cond_ref[...], wc_ref[...], preferred_element_type=jnp.float32)
    gb = gb + bc_ref[...]
    d = o_ref.shape[-1]
    gamma = gb[:, :d] + 1.0
    beta = gb[:, d:]
    o_ref[...] = jnp.maximum(hln * gamma + beta, 0.0)


def _make_gather_sum(E, D, NC, NS, C):
    NW = NC * NS
    epw = E // NW
    n_chunks = epw // C
    mesh = plsc.VectorSubcoreMesh(core_axis_name="c", subcore_axis_name="s")

    @functools.partial(
        pl.kernel,
        out_type=jax.ShapeDtypeStruct((E, D), jnp.float32),
        mesh=mesh,
        compiler_params=pltpu.CompilerParams(use_tc_tiling_on_sc=False),
        scratch_types=[
            pltpu.VMEM((C,), jnp.int32),
            pltpu.VMEM((C,), jnp.int32),
            pltpu.VMEM((C, D), jnp.float32),
            pltpu.VMEM((C, D), jnp.float32),
            pltpu.SemaphoreType.DMA,
        ],
    )
    def gather_sum(hn_hbm, src_hbm, dst_hbm, out_hbm,
                   idx_a, idx_b, rows_a, rows_b, sem):
        wid = lax.axis_index("s") * NC + lax.axis_index("c")
        base = wid * epw

        def chunk_body(i, carry):
            off = base + i * C
            pltpu.sync_copy(src_hbm.at[pl.ds(off, C)], idx_a)
            pltpu.sync_copy(dst_hbm.at[pl.ds(off, C)], idx_b)
            ca = pltpu.async_copy(hn_hbm.at[idx_a], rows_a, sem)
            cb = pltpu.async_copy(hn_hbm.at[idx_b], rows_b, sem)
            ca.wait()
            cb.wait()

            def add_body(j, c2):
                rows_a[j] = rows_a[j] + rows_b[j]
                return c2

            lax.fori_loop(0, C, add_body, 0)
            pltpu.sync_copy(rows_a, out_hbm.at[pl.ds(off, C)])
            return carry

        lax.fori_loop(0, n_chunks, chunk_body, 0)

    return gather_sum


def kernel(node_raw, edge_index, geo, cond, W_nproj, b_nproj, W_geo, b_geo,
           W_cond, b_cond, W_film, b_film):
    N = node_raw.shape[0]
    E = edge_index.shape[1]
    D = W_nproj.shape[0]
    GO = W_geo.shape[0]

    f32 = jnp.float32
    Wf1T = W_film[:, :D].T
    Wf2T = W_film[:, D:].T
    WnT = W_nproj.T
    WgT = W_geo.T
    WcT = W_cond.T

    hn = pl.pallas_call(
        _node_proj_body,
        out_shape=jax.ShapeDtypeStruct((N, D), f32),
    )(node_raw, WnT, b_nproj.reshape(1, D), Wf1T)

    src = edge_index[0].astype(jnp.int32)
    dst = edge_index[1].astype(jnp.int32)
    NC, NS = 2, 16
    g = _make_gather_sum(E, D, NC, NS, C=1000)(hn, src, dst)

    BE = 4000
    grid = E // BE
    full = lambda i: (0, 0)
    out = pl.pallas_call(
        _edge_fuse_body,
        grid=(grid,),
        in_specs=[
            pl.BlockSpec((BE, D), lambda i: (i, 0)),
            pl.BlockSpec((BE, geo.shape[1]), lambda i: (i, 0)),
            pl.BlockSpec((BE, cond.shape[1]), lambda i: (i, 0)),
            pl.BlockSpec(WgT.shape, full),
            pl.BlockSpec(Wf2T.shape, full),
            pl.BlockSpec(WcT.shape, full),
            pl.BlockSpec((1, GO), full),
            pl.BlockSpec((1, D), full),
            pl.BlockSpec((1, 2 * D), full),
        ],
        out_specs=pl.BlockSpec((BE, D), lambda i: (i, 0)),
        out_shape=jax.ShapeDtypeStruct((E, D), f32),
    )(g, geo, cond, WgT, Wf2T, WcT,
      b_geo.reshape(1, GO), b_film.reshape(1, D), b_cond.reshape(1, 2 * D))
    return out


# speedup vs baseline: 3.2960x; 1.1708x over previous
import functools

import jax
import jax.numpy as jnp
from jax import lax
from jax.experimental import pallas as pl
from jax.experimental.pallas import tpu as pltpu
from jax.experimental.pallas import tpu_sc as plsc


def _node_proj_body(x_ref, wn_ref, bn_ref, wf1_ref, o_ref):
    nf = jnp.dot(x_ref[...], wn_ref[...], preferred_element_type=jnp.float32)
    nf = jnp.maximum(nf + bn_ref[...], 0.0)
    o_ref[...] = jnp.dot(nf, wf1_ref[...], preferred_element_type=jnp.float32)


def _geo_body(x_ref, wm0_ref, wm1_ref, wm2_ref, wm3_ref, w2a_ref, w2b_ref,
              bmid_ref, bf_ref, he_ref, ho_ref):
    x = x_ref[...]
    wm = (wm0_ref[...], wm1_ref[...], wm2_ref[...], wm3_ref[...])
    m = [jnp.maximum(jnp.dot(x, w, preferred_element_type=jnp.float32)
                     + bmid_ref[...], 0.0) for w in wm]
    w2a = w2a_ref[...]
    w2b = w2b_ref[...]
    he_ref[...] = (jnp.dot(m[0], w2a, preferred_element_type=jnp.float32)
                   + jnp.dot(m[1], w2b, preferred_element_type=jnp.float32)
                   + bf_ref[...])
    ho_ref[...] = (jnp.dot(m[2], w2a, preferred_element_type=jnp.float32)
                   + jnp.dot(m[3], w2b, preferred_element_type=jnp.float32)
                   + bf_ref[...])


def _fuse_body(g_ref, cond_ref, he_ref, ho_ref, wcg_ref, wcb_ref, bdm_ref,
               bcg_ref, bcb_ref, o_ref):
    wcg = wcg_ref[...]
    wcb = wcb_ref[...]
    bdm = bdm_ref[...]
    halves = []
    for s, hg in ((slice(0, 128), he_ref[...]), (slice(128, 256), ho_ref[...])):
        h = g_ref[:, s] + hg
        c = cond_ref[:, s]
        gamma = jnp.dot(c, wcg, preferred_element_type=jnp.float32) + bcg_ref[...]
        beta = jnp.dot(c, wcb, preferred_element_type=jnp.float32) + bcb_ref[...]
        s1 = jnp.dot(h, bdm, preferred_element_type=jnp.float32)
        s2 = jnp.dot(h * h, bdm, preferred_element_type=jnp.float32)
        var = s2 - s1 * s1
        hln = (h - s1) * lax.rsqrt(var + 1e-5)
        halves.append(jnp.maximum(hln * gamma + beta, 0.0))
    o_ref[...] = jnp.concatenate(halves, axis=1)


def _make_gather_sum(E, D, NC, NS, C):
    NW = NC * NS
    epw = E // NW
    nch = epw // C
    mesh = plsc.VectorSubcoreMesh(core_axis_name="c", subcore_axis_name="s")

    @functools.partial(
        pl.kernel,
        out_type=jax.ShapeDtypeStruct((E, D), jnp.float32),
        mesh=mesh,
        compiler_params=pltpu.CompilerParams(use_tc_tiling_on_sc=False),
        scratch_types=[
            pltpu.VMEM((2, 2 * C), jnp.int32),
            pltpu.VMEM((2, 2 * C, D), jnp.float32),
            pltpu.SemaphoreType.DMA,
            pltpu.SemaphoreType.DMA,
            pltpu.SemaphoreType.DMA,
            pltpu.SemaphoreType.DMA,
        ],
    )
    def gather_sum(hn_hbm, src_hbm, dst_hbm, out_hbm,
                   idx, rows, gsem0, gsem1, wsem0, wsem1):
        gsems = (gsem0, gsem1)
        wsems = (wsem0, wsem1)
        wid = lax.axis_index("s") * NC + lax.axis_index("c")
        base = wid * epw

        def stage_and_gather(c, p):
            off = base + c * C
            pltpu.sync_copy(src_hbm.at[pl.ds(off, C)], idx.at[p, pl.ds(0, C)])
            pltpu.sync_copy(dst_hbm.at[pl.ds(off, C)], idx.at[p, pl.ds(C, C)])
            return pltpu.async_copy(hn_hbm.at[idx.at[p]], rows.at[p], gsems[p])

        gd = [None, None]
        wd = [None, None]
        gd[0] = stage_and_gather(0, 0)
        for c in range(nch):
            p = c & 1
            if c + 1 < nch:
                if wd[1 - p] is not None:
                    wd[1 - p].wait()
                gd[1 - p] = stage_and_gather(c + 1, 1 - p)
            gd[p].wait()

            def add_body(j, carry):
                rows[p, j] = rows[p, j] + rows[p, C + j]
                return carry

            lax.fori_loop(0, C, add_body, 0)
            off = base + c * C
            wd[p] = pltpu.async_copy(
                rows.at[p, pl.ds(0, C)], out_hbm.at[pl.ds(off, C)], wsems[p])
        wd[0].wait()
        wd[1].wait()

    return gather_sum


def kernel(node_raw, edge_index, geo, cond, W_nproj, b_nproj, W_geo, b_geo,
           W_cond, b_cond, W_film, b_film):
    N = node_raw.shape[0]
    E = edge_index.shape[1]
    D = W_nproj.shape[0]
    GO = W_geo.shape[0]
    GI = W_geo.shape[1]

    f32 = jnp.float32
    Wf1T = W_film[:, :D].T
    Wf2T = W_film[:, D:].T
    WnT = W_nproj.T
    WgT = W_geo.T
    WcT = W_cond.T

    hn = pl.pallas_call(
        _node_proj_body,
        out_shape=jax.ShapeDtypeStruct((N, D), f32),
    )(node_raw, WnT, b_nproj.reshape(1, D), Wf1T)

    src = edge_index[0].astype(jnp.int32)
    dst = edge_index[1].astype(jnp.int32)
    NC, NS = 2, 16
    g = _make_gather_sum(E, D, NC, NS, C=1000)(hn, src, dst)

    z = jnp.zeros((8 * D, 8 * D), f32)
    Wm = []
    for k in range(4):
        w = z
        for c in range(4):
            j = 4 * k + c
            w = w.at[GI * j:GI * (j + 1), 32 * c:32 * c + GO].set(WgT)
        Wm.append(w)
    w2a = z
    w2b = z
    for c in range(4):
        w2a = w2a.at[32 * c:32 * c + GO, D * c:D * (c + 1)].set(Wf2T)
        w2b = w2b.at[32 * c:32 * c + GO, 64 + D * c:64 + D * (c + 1)].set(Wf2T)
    bmid = jnp.tile(jnp.pad(b_geo, (0, 2)), 4).reshape(1, 128)
    bf8 = jnp.tile(b_film, 8).reshape(1, 128)

    R16 = 1000
    EG = E // 16
    geo_p = geo.reshape(EG, 128)
    full = lambda i: (0, 0)
    he, ho = pl.pallas_call(
        _geo_body,
        grid=(EG // R16,),
        in_specs=[pl.BlockSpec((R16, 128), lambda i: (i, 0))]
        + [pl.BlockSpec((128, 128), full)] * 6
        + [pl.BlockSpec((1, 128), full)] * 2,
        out_specs=[pl.BlockSpec((R16, 128), lambda i: (i, 0))] * 2,
        out_shape=[jax.ShapeDtypeStruct((EG, 128), f32)] * 2,
    )(geo_p, Wm[0], Wm[1], Wm[2], Wm[3], w2a, w2b, bmid, bf8)

    eye8 = jnp.eye(8, dtype=f32)
    bd_wcg = jnp.kron(eye8, WcT[:, :D])
    bd_wcb = jnp.kron(eye8, WcT[:, D:])
    bdm = jnp.kron(eye8, jnp.full((D, D), 1.0 / D, f32))
    bcg8 = (jnp.tile(b_cond[:D], 8) + 1.0).reshape(1, 128)
    bcb8 = jnp.tile(b_cond[D:], 8).reshape(1, 128)

    g16 = g.reshape(EG, 256)
    cond16 = cond.reshape(EG, 256)
    out16 = pl.pallas_call(
        _fuse_body,
        grid=(EG // R16,),
        in_specs=[pl.BlockSpec((R16, 256), lambda i: (i, 0)),
                  pl.BlockSpec((R16, 256), lambda i: (i, 0)),
                  pl.BlockSpec((R16, 128), lambda i: (i, 0)),
                  pl.BlockSpec((R16, 128), lambda i: (i, 0))]
        + [pl.BlockSpec((128, 128), full)] * 3
        + [pl.BlockSpec((1, 128), full)] * 2,
        out_specs=pl.BlockSpec((R16, 256), lambda i: (i, 0)),
        out_shape=jax.ShapeDtypeStruct((EG, 256), f32),
    )(g16, cond16, he, ho, bd_wcg, bd_wcb, bdm, bcg8, bcb8)
    return out16.reshape(E, D)


# speedup vs baseline: 3.5890x; 1.0889x over previous
import functools

import jax
import jax.numpy as jnp
from jax import lax
from jax.experimental import pallas as pl
from jax.experimental.pallas import tpu as pltpu
from jax.experimental.pallas import tpu_sc as plsc


def _node_proj_body(x_ref, wn_ref, bn_ref, wf1_ref, o_ref):
    nf = jnp.dot(x_ref[...], wn_ref[...], preferred_element_type=jnp.float32)
    nf = jnp.maximum(nf + bn_ref[...], 0.0)
    o_ref[...] = jnp.dot(nf, wf1_ref[...], preferred_element_type=jnp.float32)


def _fused_half(g, x, c, w):
    f32 = jnp.float32
    ma = jnp.maximum(jnp.dot(x, w["wma"], preferred_element_type=f32)
                     + w["bmid"], 0.0)
    mb = jnp.maximum(jnp.dot(x, w["wmb"], preferred_element_type=f32)
                     + w["bmid"], 0.0)
    hgeo = (jnp.dot(ma, w["w2a"], preferred_element_type=f32)
            + jnp.dot(mb, w["w2b"], preferred_element_type=f32) + w["bf8"])
    h = g + hgeo
    gamma = jnp.dot(c, w["wcg"], preferred_element_type=f32) + w["bcg8"]
    beta = jnp.dot(c, w["wcb"], preferred_element_type=f32) + w["bcb8"]
    s1 = jnp.dot(h, w["bdm"], preferred_element_type=f32)
    s2 = jnp.dot(h * h, w["bdm"], preferred_element_type=f32)
    var = s2 - s1 * s1
    hln = (h - s1) * lax.rsqrt(var + 1e-5)
    return jnp.maximum(hln * gamma + beta, 0.0)


def _fused_body(ga_ref, gb_ref, xa_ref, xb_ref, ca_ref, cb_ref,
                wma_ref, wmb_ref, w2a_ref, w2b_ref, wcg_ref, wcb_ref, bdm_ref,
                bmid_ref, bf8_ref, bcg8_ref, bcb8_ref, oa_ref, ob_ref):
    w = dict(wma=wma_ref[...], wmb=wmb_ref[...], w2a=w2a_ref[...],
             w2b=w2b_ref[...], wcg=wcg_ref[...], wcb=wcb_ref[...],
             bdm=bdm_ref[...], bmid=bmid_ref[...], bf8=bf8_ref[...],
             bcg8=bcg8_ref[...], bcb8=bcb8_ref[...])
    oa_ref[...] = _fused_half(ga_ref[...], xa_ref[...], ca_ref[...], w)
    ob_ref[...] = _fused_half(gb_ref[...], xb_ref[...], cb_ref[...], w)


def _make_gather_sum(E, D, NC, NS, C):
    NW = NC * NS
    epw = E // NW
    nch = epw // C
    mesh = plsc.VectorSubcoreMesh(core_axis_name="c", subcore_axis_name="s")

    @functools.partial(
        pl.kernel,
        out_type=jax.ShapeDtypeStruct((E, D), jnp.float32),
        mesh=mesh,
        compiler_params=pltpu.CompilerParams(use_tc_tiling_on_sc=False),
        scratch_types=[
            pltpu.VMEM((2, 2 * C), jnp.int32),
            pltpu.VMEM((2, 2 * C, D), jnp.float32),
            pltpu.SemaphoreType.DMA,
            pltpu.SemaphoreType.DMA,
            pltpu.SemaphoreType.DMA,
            pltpu.SemaphoreType.DMA,
        ],
    )
    def gather_sum(hn_hbm, src_hbm, dst_hbm, out_hbm,
                   idx, rows, gsem0, gsem1, wsem0, wsem1):
        gsems = (gsem0, gsem1)
        wsems = (wsem0, wsem1)
        wid = lax.axis_index("s") * NC + lax.axis_index("c")
        base = wid * epw

        def stage_and_gather(c, p):
            off = base + c * C
            pltpu.sync_copy(src_hbm.at[pl.ds(off, C)], idx.at[p, pl.ds(0, C)])
            pltpu.sync_copy(dst_hbm.at[pl.ds(off, C)], idx.at[p, pl.ds(C, C)])
            return pltpu.async_copy(hn_hbm.at[idx.at[p]], rows.at[p], gsems[p])

        gd = [None, None]
        wd = [None, None]
        gd[0] = stage_and_gather(0, 0)
        for c in range(nch):
            p = c & 1
            if c + 1 < nch:
                if wd[1 - p] is not None:
                    wd[1 - p].wait()
                gd[1 - p] = stage_and_gather(c + 1, 1 - p)
            gd[p].wait()

            def add_body(j, carry):
                rows[p, j] = rows[p, j] + rows[p, C + j]
                return carry

            lax.fori_loop(0, C, add_body, 0)
            off = base + c * C
            wd[p] = pltpu.async_copy(
                rows.at[p, pl.ds(0, C)], out_hbm.at[pl.ds(off, C)], wsems[p])
        wd[0].wait()
        wd[1].wait()

    return gather_sum


def kernel(node_raw, edge_index, geo, cond, W_nproj, b_nproj, W_geo, b_geo,
           W_cond, b_cond, W_film, b_film):
    N = node_raw.shape[0]
    E = edge_index.shape[1]
    D = W_nproj.shape[0]
    GO = W_geo.shape[0]
    GI = W_geo.shape[1]

    f32 = jnp.float32
    Wf1T = W_film[:, :D].T
    Wf2T = W_film[:, D:].T
    WnT = W_nproj.T
    WgT = W_geo.T
    WcT = W_cond.T

    hn = pl.pallas_call(
        _node_proj_body,
        out_shape=jax.ShapeDtypeStruct((N, D), f32),
    )(node_raw, WnT, b_nproj.reshape(1, D), Wf1T)

    src = edge_index[0].astype(jnp.int32)
    dst = edge_index[1].astype(jnp.int32)
    NC, NS = 2, 16
    g = _make_gather_sum(E, D, NC, NS, C=1000)(hn, src, dst)

    wma = jnp.zeros((64, 128), f32)
    wmb = jnp.zeros((64, 128), f32)
    w2a = jnp.zeros((128, 128), f32)
    w2b = jnp.zeros((128, 128), f32)
    for c in range(4):
        wma = wma.at[GI * c:GI * (c + 1), 32 * c:32 * c + GO].set(WgT)
        wmb = wmb.at[32 + GI * c:32 + GI * (c + 1), 32 * c:32 * c + GO].set(WgT)
        w2a = w2a.at[32 * c:32 * c + GO, D * c:D * (c + 1)].set(Wf2T)
        w2b = w2b.at[32 * c:32 * c + GO, 64 + D * c:64 + D * (c + 1)].set(Wf2T)
    bmid = jnp.tile(jnp.pad(b_geo, (0, 2)), 4).reshape(1, 128)
    bf8 = jnp.tile(b_film, 8).reshape(1, 128)
    eye8 = jnp.eye(8, dtype=f32)
    bd_wcg = jnp.kron(eye8, WcT[:, :D])
    bd_wcb = jnp.kron(eye8, WcT[:, D:])
    bdm = jnp.kron(eye8, jnp.full((D, D), 1.0 / D, f32))
    bcg8 = (jnp.tile(b_cond[:D], 8) + 1.0).reshape(1, 128)
    bcb8 = jnp.tile(b_cond[D:], 8).reshape(1, 128)

    EG8 = E // 8
    g8 = g.reshape(EG8, 128)
    cond8 = cond.reshape(EG8, 128)
    geo8 = geo.reshape(EG8, 64)

    R8 = 1000
    G2 = EG8 // R8 // 2
    half = EG8 // 2
    hr = half // R8
    full = lambda i: (0, 0)
    band_a = lambda i: (i, 0)
    band_b = lambda i: (i + hr, 0)
    out_a, out_b = pl.pallas_call(
        _fused_body,
        grid=(G2,),
        in_specs=[pl.BlockSpec((R8, 128), band_a),
                  pl.BlockSpec((R8, 128), band_b),
                  pl.BlockSpec((R8, 64), band_a),
                  pl.BlockSpec((R8, 64), band_b),
                  pl.BlockSpec((R8, 128), band_a),
                  pl.BlockSpec((R8, 128), band_b)]
        + [pl.BlockSpec((64, 128), full)] * 2
        + [pl.BlockSpec((128, 128), full)] * 5
        + [pl.BlockSpec((1, 128), full)] * 4,
        out_specs=[pl.BlockSpec((R8, 128), band_a)] * 2,
        out_shape=[jax.ShapeDtypeStruct((half, 128), f32)] * 2,
    )(g8, g8, geo8, geo8, cond8, cond8,
      wma, wmb, w2a, w2b, bd_wcg, bd_wcb, bdm, bmid, bf8, bcg8, bcb8)
    return jnp.concatenate([out_a, out_b], axis=0).reshape(E, D)


# speedup vs baseline: 3.7558x; 1.0465x over previous
import functools

import jax
import jax.numpy as jnp
from jax import lax
from jax.experimental import pallas as pl
from jax.experimental.pallas import tpu as pltpu
from jax.experimental.pallas import tpu_sc as plsc


def _node_proj_body(x_ref, wn_ref, bn_ref, wf1_ref, o_ref):
    nf = jnp.dot(x_ref[...], wn_ref[...], preferred_element_type=jnp.float32)
    nf = jnp.maximum(nf + bn_ref[...], 0.0)
    o_ref[...] = jnp.dot(nf, wf1_ref[...], preferred_element_type=jnp.float32)


def _fused_half(g, x, c, w):
    f32 = jnp.float32
    ma = jnp.maximum(jnp.dot(x, w["wma"], preferred_element_type=f32)
                     + w["bmid"], 0.0)
    mb = jnp.maximum(jnp.dot(x, w["wmb"], preferred_element_type=f32)
                     + w["bmid"], 0.0)
    hgeo = (jnp.dot(ma, w["w2a"], preferred_element_type=f32)
            + jnp.dot(mb, w["w2b"], preferred_element_type=f32) + w["bf8"])
    h = g + hgeo
    gamma = jnp.dot(c, w["wcg"], preferred_element_type=f32) + w["bcg8"]
    beta = jnp.dot(c, w["wcb"], preferred_element_type=f32) + w["bcb8"]
    s1 = jnp.dot(h, w["bdm"], preferred_element_type=f32)
    s2 = jnp.dot(h * h, w["bdm"], preferred_element_type=f32)
    var = s2 - s1 * s1
    hln = (h - s1) * lax.rsqrt(var + 1e-5)
    return jnp.maximum(hln * gamma + beta, 0.0)


def _fused_body(ga_ref, gb_ref, xa_ref, xb_ref, ca_ref, cb_ref,
                wma_ref, wmb_ref, w2a_ref, w2b_ref, wcg_ref, wcb_ref, bdm_ref,
                bmid_ref, bf8_ref, bcg8_ref, bcb8_ref, o_ref):
    w = dict(wma=wma_ref[...], wmb=wmb_ref[...], w2a=w2a_ref[...],
             w2b=w2b_ref[...], wcg=wcg_ref[...], wcb=wcb_ref[...],
             bdm=bdm_ref[...], bmid=bmid_ref[...], bf8=bf8_ref[...],
             bcg8=bcg8_ref[...], bcb8=bcb8_ref[...])
    o_ref[0] = _fused_half(ga_ref[...], xa_ref[...], ca_ref[...], w)
    o_ref[1] = _fused_half(gb_ref[...], xb_ref[...], cb_ref[...], w)


def _make_gather_sum(E, D, NC, NS, C):
    NW = NC * NS
    epw = E // NW
    nch = epw // C
    mesh = plsc.VectorSubcoreMesh(core_axis_name="c", subcore_axis_name="s")

    @functools.partial(
        pl.kernel,
        out_type=jax.ShapeDtypeStruct((E, D), jnp.float32),
        mesh=mesh,
        compiler_params=pltpu.CompilerParams(use_tc_tiling_on_sc=False),
        scratch_types=[
            pltpu.VMEM((2, 2 * C), jnp.int32),
            pltpu.VMEM((2, 2 * C, D), jnp.float32),
            pltpu.SemaphoreType.DMA,
            pltpu.SemaphoreType.DMA,
            pltpu.SemaphoreType.DMA,
            pltpu.SemaphoreType.DMA,
        ],
    )
    def gather_sum(hn_hbm, ei_hbm, out_hbm,
                   idx, rows, gsem0, gsem1, wsem0, wsem1):
        gsems = (gsem0, gsem1)
        wsems = (wsem0, wsem1)
        wid = lax.axis_index("s") * NC + lax.axis_index("c")
        base = wid * epw

        def stage_and_gather(c, p):
            off = base + c * C
            pltpu.sync_copy(ei_hbm.at[0, pl.ds(off, C)], idx.at[p, pl.ds(0, C)])
            pltpu.sync_copy(ei_hbm.at[1, pl.ds(off, C)], idx.at[p, pl.ds(C, C)])
            return pltpu.async_copy(hn_hbm.at[idx.at[p]], rows.at[p], gsems[p])

        gd = [None, None]
        wd = [None, None]
        gd[0] = stage_and_gather(0, 0)
        for c in range(nch):
            p = c & 1
            if c + 1 < nch:
                if wd[1 - p] is not None:
                    wd[1 - p].wait()
                gd[1 - p] = stage_and_gather(c + 1, 1 - p)
            gd[p].wait()

            def add_body(j, carry):
                rows[p, j] = rows[p, j] + rows[p, C + j]
                return carry

            lax.fori_loop(0, C, add_body, 0)
            off = base + c * C
            wd[p] = pltpu.async_copy(
                rows.at[p, pl.ds(0, C)], out_hbm.at[pl.ds(off, C)], wsems[p])
        wd[0].wait()
        wd[1].wait()

    return gather_sum


def kernel(node_raw, edge_index, geo, cond, W_nproj, b_nproj, W_geo, b_geo,
           W_cond, b_cond, W_film, b_film):
    N = node_raw.shape[0]
    E = edge_index.shape[1]
    D = W_nproj.shape[0]
    GO = W_geo.shape[0]
    GI = W_geo.shape[1]

    f32 = jnp.float32
    Wf1T = W_film[:, :D].T
    Wf2T = W_film[:, D:].T
    WnT = W_nproj.T
    WgT = W_geo.T
    WcT = W_cond.T

    hn = pl.pallas_call(
        _node_proj_body,
        out_shape=jax.ShapeDtypeStruct((N, D), f32),
    )(node_raw, WnT, b_nproj.reshape(1, D), Wf1T)

    NC, NS = 2, 16
    g = _make_gather_sum(E, D, NC, NS, C=1000)(
        hn, edge_index.astype(jnp.int32))

    wma = jnp.zeros((64, 128), f32)
    wmb = jnp.zeros((64, 128), f32)
    w2a = jnp.zeros((128, 128), f32)
    w2b = jnp.zeros((128, 128), f32)
    for c in range(4):
        wma = wma.at[GI * c:GI * (c + 1), 32 * c:32 * c + GO].set(WgT)
        wmb = wmb.at[32 + GI * c:32 + GI * (c + 1), 32 * c:32 * c + GO].set(WgT)
        w2a = w2a.at[32 * c:32 * c + GO, D * c:D * (c + 1)].set(Wf2T)
        w2b = w2b.at[32 * c:32 * c + GO, 64 + D * c:64 + D * (c + 1)].set(Wf2T)
    bmid = jnp.tile(jnp.pad(b_geo, (0, 2)), 4).reshape(1, 128)
    bf8 = jnp.tile(b_film, 8).reshape(1, 128)
    eye8 = jnp.eye(8, dtype=f32)
    bd_wcg = jnp.kron(eye8, WcT[:, :D])
    bd_wcb = jnp.kron(eye8, WcT[:, D:])
    bdm = jnp.kron(eye8, jnp.full((D, D), 1.0 / D, f32))
    bcg8 = (jnp.tile(b_cond[:D], 8) + 1.0).reshape(1, 128)
    bcb8 = jnp.tile(b_cond[D:], 8).reshape(1, 128)

    EG8 = E // 8
    g8 = g.reshape(EG8, 128)
    cond8 = cond.reshape(EG8, 128)
    geo8 = geo.reshape(EG8, 64)

    R8 = 1000
    G2 = EG8 // R8 // 2
    half = EG8 // 2
    hr = half // R8
    full = lambda i: (0, 0)
    band_a = lambda i: (i, 0)
    band_b = lambda i: (i + hr, 0)
    out = pl.pallas_call(
        _fused_body,
        grid=(G2,),
        in_specs=[pl.BlockSpec((R8, 128), band_a),
                  pl.BlockSpec((R8, 128), band_b),
                  pl.BlockSpec((R8, 64), band_a),
                  pl.BlockSpec((R8, 64), band_b),
                  pl.BlockSpec((R8, 128), band_a),
                  pl.BlockSpec((R8, 128), band_b)]
        + [pl.BlockSpec((64, 128), full)] * 2
        + [pl.BlockSpec((128, 128), full)] * 5
        + [pl.BlockSpec((1, 128), full)] * 4,
        out_specs=pl.BlockSpec((2, R8, 128), lambda i: (0, i, 0)),
        out_shape=jax.ShapeDtypeStruct((2, half, 128), f32),
    )(g8, g8, geo8, geo8, cond8, cond8,
      wma, wmb, w2a, w2b, bd_wcg, bd_wcb, bdm, bmid, bf8, bcg8, bcb8)
    return out.reshape(E, D)
